# baseline (speedup 1.0000x reference)
import jax
import jax.numpy as jnp
from jax.experimental import pallas as pl
from jax.experimental.pallas import tpu as pltpu

B = 1024
H = 32
SELF_DIM = 6
AGENT_DIM = 7
HID = 50
OUT = 32
P = 4
BB = 256
RB = BB * H // P


def _split(w):
    hi = w.astype(jnp.bfloat16)
    lo = (w - hi.astype(jnp.float32)).astype(jnp.bfloat16)
    return hi, lo


def _bdiag(w):
    r, c = w.shape
    z = jnp.zeros((r, c), w.dtype)
    rows = [jnp.concatenate([w if i == j else z for j in range(P)], axis=1)
            for i in range(P)]
    return jnp.concatenate(rows, axis=0)


def _fused(xs_ref, xh_ref,
           wr1, br1, wr2, br2,
           w1h, w1l, b1t, w2h, w2l, b2t, wch, wcl,
           rel_rh, rel_hh, b_t,
           rel_hr, root_hr, b_hr,
           rel2, root2, b2,
           wv1, bv1, wv2, bv2, wv3, bv3, wv4, bv4,
           out_ref):
    dot = lambda a, b: jax.lax.dot(a, b, preferred_element_type=jnp.float32,
                                   precision=jax.lax.Precision.HIGHEST)
    d1 = lambda u, v: jax.lax.dot(u, v, preferred_element_type=jnp.float32)

    def dot3(a, bh, bl):
        ah, al = _split(a)
        return d1(ah, bh[...]) + d1(ah, bl[...]) + d1(al, bh[...])

    relu = lambda x: jnp.maximum(x, 0.0)
    xs = xs_ref[...]
    xh = xh_ref[...]
    r_emb = relu(dot(relu(dot(xs, wr1[...]) + br1[...]), wr2[...]) + br2[...])
    h1 = relu(dot3(xh, w1h, w1l) + b1t[...])
    h_emb = relu(dot3(h1, w2h, w2l) + b2t[...])
    e = jnp.sum(h_emb.reshape(BB, H // P, P * OUT), axis=1)
    s1 = (e[:, 0:OUT] + e[:, OUT:2 * OUT]
          + e[:, 2 * OUT:3 * OUT] + e[:, 3 * OUT:4 * OUT])
    t = (dot(r_emb, rel_rh[...]) + dot(s1 * (1.0 / (H - 1)), rel_hh[...])
         + b_t[...])
    tt = jnp.concatenate([t] * P, axis=1)
    m = dot3(h_emb, wch, wcl)
    sm = jnp.sum(relu(m.reshape(BB, H // P, P * HID)
                      + tt[:, None, :]), axis=1)
    s2 = (sm[:, 0:HID] + sm[:, HID:2 * HID]
          + sm[:, 2 * HID:3 * HID] + sm[:, 3 * HID:4 * HID])
    h_rob = relu(dot(s1 * (1.0 / H), rel_hr[...]) + dot(r_emb, root_hr[...])
                 + b_hr[...])
    h2 = relu(dot(s2 * (1.0 / H), rel2[...]) + dot(h_rob, root2[...]) + b2[...])
    v = relu(dot(h2, wv1[...]) + bv1[...])
    v = relu(dot(v, wv2[...]) + bv2[...])
    v = relu(dot(v, wv3[...]) + bv3[...])
    out_ref[...] = dot(v, wv4[...]) + bv4[...]


def kernel(state, dropout, params, ei_rh, ei_hr, ei_hh):
    p = params
    (wr1, br1), (wr2, br2) = p['w_r']
    (wh1, bh1), (wh2, bh2) = p['w_h']
    rel_rh, root_rh, b_rh = p['conv1_rh']
    rel_hh, root_hh, b_hh = p['conv1_hh']
    rel_hr, root_hr, b_hr = p['conv1_hr']
    rel2, root2, b2 = p['conv2_hr']
    (wv1, bv1), (wv2, bv2), (wv3, bv3), (wv4, bv4) = p['value']
    xs = state[:, 0, :SELF_DIM]
    xh = state[:, :, SELF_DIM:].reshape(B * H // P, P * AGENT_DIM)
    wc = root_rh + root_hh - rel_hh * (1.0 / (H - 1))
    w1h, w1l = _split(_bdiag(wh1))
    w2h, w2l = _split(_bdiag(wh2))
    wch, wcl = _split(_bdiag(wc))
    r2 = lambda v: v.reshape(1, -1)
    tile = lambda v: jnp.concatenate([r2(v)] * P, axis=1)
    weights = [wr1, r2(br1), wr2, r2(br2),
               w1h, w1l, tile(bh1), w2h, w2l, tile(bh2), wch, wcl,
               rel_rh, rel_hh, r2(b_rh + b_hh),
               rel_hr, root_hr, r2(b_hr),
               rel2, root2, r2(b2),
               wv1, r2(bv1), wv2, r2(bv2), wv3, r2(bv3), wv4, r2(bv4)]
    full = lambda w: pl.BlockSpec(w.shape, lambda i: (0, 0))
    grid = B // BB
    out = pl.pallas_call(
        _fused,
        grid=(grid,),
        in_specs=[pl.BlockSpec((BB, SELF_DIM), lambda i: (i, 0)),
                  pl.BlockSpec((RB, P * AGENT_DIM), lambda i: (i, 0))]
                 + [full(w) for w in weights],
        out_specs=pl.BlockSpec((BB, 1), lambda i: (i, 0)),
        out_shape=jax.ShapeDtypeStruct((B, 1), jnp.float32),
        compiler_params=pltpu.CompilerParams(
            dimension_semantics=("parallel",)),
    )(xs, xh, *weights)
    return out


# speedup vs baseline: 1.0887x; 1.0887x over previous
import jax
import jax.numpy as jnp
from jax.experimental import pallas as pl
from jax.experimental.pallas import tpu as pltpu

B = 1024
H = 32
SELF_DIM = 6
AGENT_DIM = 7
HID = 50
OUT = 32
P = 4
BB = 256
RB = BB * H // P


def _split(w):
    hi = w.astype(jnp.bfloat16)
    lo = (w - hi.astype(jnp.float32)).astype(jnp.bfloat16)
    return hi, lo


def _bdiag(w):
    r, c = w.shape
    z = jnp.zeros((r, c), w.dtype)
    rows = [jnp.concatenate([w if i == j else z for j in range(P)], axis=1)
            for i in range(P)]
    return jnp.concatenate(rows, axis=0)


def _fused(xs_ref, xh_ref,
           wr1, br1, wr2, br2,
           wh1, bh1, wh2, bh2,
           rel_rh, root_rh, b_rh,
           rel_hh, root_hh, b_hh,
           rel_hr, root_hr, b_hr,
           rel2, root2, b2,
           wv1, bv1, wv2, bv2, wv3, bv3, wv4, bv4,
           out_ref,
           w1h_s, w1l_s, w2h_s, w2l_s, wch_s, wcl_s):
    dot = lambda a, b: jax.lax.dot(a, b, preferred_element_type=jnp.float32,
                                   precision=jax.lax.Precision.HIGHEST)
    d1 = lambda u, v: jax.lax.dot(u, v, preferred_element_type=jnp.float32)

    @pl.when(pl.program_id(0) == 0)
    def _build_weights():
        w1h, w1l = _split(_bdiag(wh1[...]))
        w2h, w2l = _split(_bdiag(wh2[...]))
        wc = root_rh[...] + root_hh[...] - rel_hh[...] * (1.0 / (H - 1))
        wch, wcl = _split(_bdiag(wc))
        w1h_s[...] = w1h
        w1l_s[...] = w1l
        w2h_s[...] = w2h
        w2l_s[...] = w2l
        wch_s[...] = wch
        wcl_s[...] = wcl

    def dot3(a, bh_s, bl_s):
        ah, al = _split(a)
        bh, bl = bh_s[...], bl_s[...]
        return d1(ah, bh) + d1(ah, bl) + d1(al, bh)

    relu = lambda x: jnp.maximum(x, 0.0)
    xs = xs_ref[...]
    xh = xh_ref[...]
    r_emb = relu(dot(relu(dot(xs, wr1[...]) + br1[...]), wr2[...]) + br2[...])
    b1t = jnp.concatenate([bh1[...]] * P, axis=1)
    b2t = jnp.concatenate([bh2[...]] * P, axis=1)
    h1 = relu(dot3(xh, w1h_s, w1l_s) + b1t)
    h_emb = relu(dot3(h1, w2h_s, w2l_s) + b2t)
    e = jnp.sum(h_emb.reshape(BB, H // P, P * OUT), axis=1)
    s1 = (e[:, 0:OUT] + e[:, OUT:2 * OUT]
          + e[:, 2 * OUT:3 * OUT] + e[:, 3 * OUT:4 * OUT])
    t = (dot(r_emb, rel_rh[...]) + dot(s1 * (1.0 / (H - 1)), rel_hh[...])
         + b_rh[...] + b_hh[...])
    tt = jnp.concatenate([t] * P, axis=1)
    m = dot3(h_emb, wch_s, wcl_s)
    sm = jnp.sum(relu(m.reshape(BB, H // P, P * HID)
                      + tt[:, None, :]), axis=1)
    s2 = (sm[:, 0:HID] + sm[:, HID:2 * HID]
          + sm[:, 2 * HID:3 * HID] + sm[:, 3 * HID:4 * HID])
    h_rob = relu(dot(s1 * (1.0 / H), rel_hr[...]) + dot(r_emb, root_hr[...])
                 + b_hr[...])
    h2 = relu(dot(s2 * (1.0 / H), rel2[...]) + dot(h_rob, root2[...]) + b2[...])
    v = relu(dot(h2, wv1[...]) + bv1[...])
    v = relu(dot(v, wv2[...]) + bv2[...])
    v = relu(dot(v, wv3[...]) + bv3[...])
    out_ref[...] = dot(v, wv4[...]) + bv4[...]


def kernel(state, dropout, params, ei_rh, ei_hr, ei_hh):
    p = params
    (wr1, br1), (wr2, br2) = p['w_r']
    (wh1, bh1), (wh2, bh2) = p['w_h']
    rel_rh, root_rh, b_rh = p['conv1_rh']
    rel_hh, root_hh, b_hh = p['conv1_hh']
    rel_hr, root_hr, b_hr = p['conv1_hr']
    rel2, root2, b2 = p['conv2_hr']
    (wv1, bv1), (wv2, bv2), (wv3, bv3), (wv4, bv4) = p['value']
    xs = state[:, 0, :SELF_DIM]
    xh = state[:, :, SELF_DIM:].reshape(B * H // P, P * AGENT_DIM)
    r2 = lambda v: v.reshape(1, -1)
    weights = [wr1, r2(br1), wr2, r2(br2),
               wh1, r2(bh1), wh2, r2(bh2),
               rel_rh, root_rh, r2(b_rh),
               rel_hh, root_hh, r2(b_hh),
               rel_hr, root_hr, r2(b_hr),
               rel2, root2, r2(b2),
               wv1, r2(bv1), wv2, r2(bv2), wv3, r2(bv3), wv4, r2(bv4)]
    full = lambda w: pl.BlockSpec(w.shape, lambda i: (0, 0))
    grid = B // BB
    bf = jnp.bfloat16
    out = pl.pallas_call(
        _fused,
        grid=(grid,),
        in_specs=[pl.BlockSpec((BB, SELF_DIM), lambda i: (i, 0)),
                  pl.BlockSpec((RB, P * AGENT_DIM), lambda i: (i, 0))]
                 + [full(w) for w in weights],
        out_specs=pl.BlockSpec((BB, 1), lambda i: (i, 0)),
        out_shape=jax.ShapeDtypeStruct((B, 1), jnp.float32),
        scratch_shapes=[pltpu.VMEM((P * AGENT_DIM, P * 64), bf),
                        pltpu.VMEM((P * AGENT_DIM, P * 64), bf),
                        pltpu.VMEM((P * 64, P * OUT), bf),
                        pltpu.VMEM((P * 64, P * OUT), bf),
                        pltpu.VMEM((P * OUT, P * HID), bf),
                        pltpu.VMEM((P * OUT, P * HID), bf)],
        compiler_params=pltpu.CompilerParams(
            dimension_semantics=("arbitrary",)),
    )(xs, xh, *weights)
    return out


# speedup vs baseline: 1.0952x; 1.0060x over previous
import jax
import jax.numpy as jnp
from jax.experimental import pallas as pl
from jax.experimental.pallas import tpu as pltpu

B = 1024
H = 32
IN_DIM = 13
SELF_DIM = 6
AGENT_DIM = 7
HID = 50
OUT = 32
P = 4
BB = 256
RB = BB * H // P


def _split(w):
    hi = w.astype(jnp.bfloat16)
    lo = (w - hi.astype(jnp.float32)).astype(jnp.bfloat16)
    return hi, lo


def _bdiag(w):
    r, c = w.shape
    z = jnp.zeros((r, c), w.dtype)
    rows = [jnp.concatenate([w if i == j else z for j in range(P)], axis=1)
            for i in range(P)]
    return jnp.concatenate(rows, axis=0)


def _fused(xs_ref, xg_ref,
           wr1, br1, wr2, br2,
           wh1, bh1, wh2, bh2,
           rel_rh, root_rh, b_rh,
           rel_hh, root_hh, b_hh,
           rel_hr, root_hr, b_hr,
           rel2, root2, b2,
           wv1, bv1, wv2, bv2, wv3, bv3, wv4, bv4,
           out_ref,
           w1h_s, w1l_s, w2h_s, w2l_s, wch_s, wcl_s):
    dot = lambda a, b: jax.lax.dot(a, b, preferred_element_type=jnp.float32,
                                   precision=jax.lax.Precision.HIGHEST)
    d1 = lambda u, v: jax.lax.dot(u, v, preferred_element_type=jnp.float32)

    @pl.when(pl.program_id(0) == 0)
    def _build_weights():
        wh1p = jnp.concatenate(
            [jnp.zeros((SELF_DIM, 64), jnp.float32), wh1[...]], axis=0)
        w1h, w1l = _split(_bdiag(wh1p))
        w2h, w2l = _split(_bdiag(wh2[...]))
        wc = root_rh[...] + root_hh[...] - rel_hh[...] * (1.0 / (H - 1))
        wch, wcl = _split(_bdiag(wc))
        w1h_s[...] = w1h
        w1l_s[...] = w1l
        w2h_s[...] = w2h
        w2l_s[...] = w2l
        wch_s[...] = wch
        wcl_s[...] = wcl

    def dot3(a, bh_s, bl_s):
        ah, al = _split(a)
        bh, bl = bh_s[...], bl_s[...]
        return d1(ah, bh) + d1(ah, bl) + d1(al, bh)

    relu = lambda x: jnp.maximum(x, 0.0)
    row = lambda v: v[...][None, :]
    xs = xs_ref[...]
    xg = xg_ref[...]
    r_emb = relu(dot(relu(dot(xs, wr1[...]) + row(br1)), wr2[...]) + row(br2))
    b1t = jnp.concatenate([row(bh1)] * P, axis=1)
    b2t = jnp.concatenate([row(bh2)] * P, axis=1)
    h1 = relu(dot3(xg, w1h_s, w1l_s) + b1t)
    h_emb = relu(dot3(h1, w2h_s, w2l_s) + b2t)
    e = jnp.sum(h_emb.reshape(BB, H // P, P * OUT), axis=1)
    s1 = (e[:, 0:OUT] + e[:, OUT:2 * OUT]
          + e[:, 2 * OUT:3 * OUT] + e[:, 3 * OUT:4 * OUT])
    t = (dot(r_emb, rel_rh[...]) + dot(s1 * (1.0 / (H - 1)), rel_hh[...])
         + row(b_rh) + row(b_hh))
    tt = jnp.concatenate([t] * P, axis=1)
    m = dot3(h_emb, wch_s, wcl_s)
    sm = jnp.sum(relu(m.reshape(BB, H // P, P * HID)
                      + tt[:, None, :]), axis=1)
    s2 = (sm[:, 0:HID] + sm[:, HID:2 * HID]
          + sm[:, 2 * HID:3 * HID] + sm[:, 3 * HID:4 * HID])
    h_rob = relu(dot(s1 * (1.0 / H), rel_hr[...]) + dot(r_emb, root_hr[...])
                 + row(b_hr))
    h2 = relu(dot(s2 * (1.0 / H), rel2[...]) + dot(h_rob, root2[...]) + row(b2))
    v = relu(dot(h2, wv1[...]) + row(bv1))
    v = relu(dot(v, wv2[...]) + row(bv2))
    v = relu(dot(v, wv3[...]) + row(bv3))
    out_ref[...] = dot(v, wv4[...]) + row(bv4)


def kernel(state, dropout, params, ei_rh, ei_hr, ei_hh):
    p = params
    (wr1, br1), (wr2, br2) = p['w_r']
    (wh1, bh1), (wh2, bh2) = p['w_h']
    rel_rh, root_rh, b_rh = p['conv1_rh']
    rel_hh, root_hh, b_hh = p['conv1_hh']
    rel_hr, root_hr, b_hr = p['conv1_hr']
    rel2, root2, b2 = p['conv2_hr']
    (wv1, bv1), (wv2, bv2), (wv3, bv3), (wv4, bv4) = p['value']
    xs = state[:, 0, :SELF_DIM]
    xg = state.reshape(B * H // P, P * IN_DIM)
    weights = [wr1, br1, wr2, br2,
               wh1, bh1, wh2, bh2,
               rel_rh, root_rh, b_rh,
               rel_hh, root_hh, b_hh,
               rel_hr, root_hr, b_hr,
               rel2, root2, b2,
               wv1, bv1, wv2, bv2, wv3, bv3, wv4, bv4]
    full = lambda w: pl.BlockSpec(w.shape, (lambda i: (0, 0)) if w.ndim == 2
                                  else (lambda i: (0,)))
    grid = B // BB
    bf = jnp.bfloat16
    out = pl.pallas_call(
        _fused,
        grid=(grid,),
        in_specs=[pl.BlockSpec((BB, SELF_DIM), lambda i: (i, 0)),
                  pl.BlockSpec((RB, P * IN_DIM), lambda i: (i, 0))]
                 + [full(w) for w in weights],
        out_specs=pl.BlockSpec((BB, 1), lambda i: (i, 0)),
        out_shape=jax.ShapeDtypeStruct((B, 1), jnp.float32),
        scratch_shapes=[pltpu.VMEM((P * IN_DIM, P * 64), bf),
                        pltpu.VMEM((P * IN_DIM, P * 64), bf),
                        pltpu.VMEM((P * 64, P * OUT), bf),
                        pltpu.VMEM((P * 64, P * OUT), bf),
                        pltpu.VMEM((P * OUT, P * HID), bf),
                        pltpu.VMEM((P * OUT, P * HID), bf)],
        compiler_params=pltpu.CompilerParams(
            dimension_semantics=("arbitrary",)),
    )(xs, xg, *weights)
    return out
